# Initial kernel scaffold; baseline (speedup 1.0000x reference)
import jax
import jax.numpy as jnp
from jax.experimental import pallas as pl


def kernel(x, edge_index, edge_attr, W1, b1, W2, b2, W3, b3, W4, b4):
    raise NotImplementedError("write your pallas kernel here")


# speedup vs baseline: 1.6720x; 1.6720x over previous
import functools

import jax
import jax.numpy as jnp
from jax import lax
from jax.experimental import pallas as pl
from jax.experimental.pallas import tpu as pltpu
from jax.experimental.pallas import tpu_sc as plsc

D = 128
NV = D // 16
N = 10000
E = 320000
NC = 2
NS = 16
EPT = E // NS
BB = 80
NB = EPT // BB
NR = 5120
TRASH = NR
SR = NR + 8
RPT = NR // NS
OC = 16
CW = 16


def _proj_body(x_ref, w1_ref, xa_ref, xb_ref):
    xv = x_ref[...]
    xa_ref[...] = jnp.dot(xv, w1_ref[0:D, :], preferred_element_type=jnp.float32)
    xb_ref[...] = jnp.dot(xv, w1_ref[D:2 * D, :], preferred_element_type=jnp.float32)


def _edge_loop_body(xa, xb, attr, w1cb, src_i, dst_i,
                    s_out, c_out,
                    idx_s, idx_d, idx_dl, buf_a, buf_b, abuf, wbuf,
                    ones_v, zbuf_c, s_sh, c_sh, sem_a, sem_b):
    cid = lax.axis_index("c")
    sid = lax.axis_index("s")

    zero16 = jnp.zeros((16,), jnp.float32)
    onerow = jnp.where(lax.iota(jnp.int32, 16) == 0,
                       jnp.float32(1.0), jnp.float32(0.0))

    def _zrow(i, _):
        for v in range(NV):
            buf_a[i, pl.ds(v * 16, 16)] = zero16
        zbuf_c[i, :] = zero16
        return 0
    lax.fori_loop(0, OC, _zrow, 0)

    def _orow(i, _):
        ones_v[i, :] = onerow
        return 0
    lax.fori_loop(0, BB, _orow, 0)

    def _zcp(i, _):
        row0 = sid * RPT + i * OC
        pltpu.sync_copy(buf_a.at[pl.ds(0, OC)], s_sh.at[pl.ds(row0, OC)])
        pltpu.sync_copy(zbuf_c.at[pl.ds(0, OC)], c_sh.at[pl.ds(row0, OC)])
        return 0
    lax.fori_loop(0, RPT // OC, _zcp, 0)
    plsc.subcore_barrier()

    roff = cid * NR
    trash16 = jnp.full((16,), TRASH, jnp.int32)

    pltpu.sync_copy(w1cb, wbuf)
    wv = [[wbuf[k, pl.ds(v * 16, 16)] for k in range(5)] for v in range(NV)]

    base = sid * EPT

    def _batch(j, _):
        eoff = base + j * BB
        pltpu.sync_copy(attr.at[pl.ds(eoff * 4, BB * 4)], abuf)
        pltpu.sync_copy(src_i.at[pl.ds(eoff, BB)], idx_s)
        pltpu.sync_copy(dst_i.at[pl.ds(eoff, BB)], idx_d)
        cp_a = pltpu.async_copy(xa.at[idx_s], buf_a, sem_a)
        cp_b = pltpu.async_copy(xb.at[idx_d], buf_b, sem_b)
        for v in range(BB // 16):
            sl = pl.ds(v * 16, 16)
            lidx = idx_d[sl] - roff
            ok = (lidx >= 0) & (lidx < NR)
            idx_dl[0, sl] = jnp.where(ok, lidx, trash16)
        cp_a.wait()
        cp_b.wait()

        def _rowq(q, _):
            av = abuf[pl.ds(16 * q, 16)]
            for p in range(4):
                r = 4 * q + p
                s0, s1, s2, s3 = (av[4 * p], av[4 * p + 1],
                                  av[4 * p + 2], av[4 * p + 3])
                for v in range(NV):
                    sl = pl.ds(v * 16, 16)
                    t = (wv[v][4] + s0 * wv[v][0] + s1 * wv[v][1]
                         + s2 * wv[v][2] + s3 * wv[v][3])
                    buf_a[r, sl] = jnp.maximum(
                        buf_a[r, sl] + buf_b[r, sl] + t, jnp.float32(0.0))
            return 0
        lax.fori_loop(0, BB // 4, _rowq, 0)

        pltpu.sync_copy(buf_a, s_sh.at[idx_dl.at[0]], add=True)
        pltpu.sync_copy(ones_v, c_sh.at[idx_dl.at[0]], add=True)
        return 0
    lax.fori_loop(0, NB, _batch, 0)
    plsc.subcore_barrier()

    def _ocp(i, _):
        row0 = sid * RPT + i * OC
        pltpu.sync_copy(s_sh.at[pl.ds(row0, OC)], s_out.at[cid, pl.ds(row0, OC)])
        pltpu.sync_copy(c_sh.at[pl.ds(row0, OC)], c_out.at[cid, pl.ds(row0, OC)])
        return 0
    lax.fori_loop(0, RPT // OC, _ocp, 0)


def _node_body(x_ref, s_ref, c_ref, w2_ref, b2_ref, w3_ref, b3_ref,
               w4_ref, b4_ref, out_ref):
    xv = x_ref[...]
    cnt = c_ref[:, 0:1]
    agg = (jnp.dot(s_ref[...], w2_ref[...], preferred_element_type=jnp.float32)
           + cnt * b2_ref[...])
    hn = jnp.maximum(
        jnp.dot(xv, w3_ref[0:D, :], preferred_element_type=jnp.float32)
        + jnp.dot(agg, w3_ref[D:2 * D, :], preferred_element_type=jnp.float32)
        + b3_ref[...],
        jnp.float32(0.0))
    out_ref[...] = (jnp.dot(hn, w4_ref[...], preferred_element_type=jnp.float32)
                    + b4_ref[...] + xv)


def kernel(x, edge_index, edge_attr, W1, b1, W2, b2, W3, b3, W4, b4):
    src = edge_index[0].astype(jnp.int32)
    dst = edge_index[1].astype(jnp.int32)
    attr_flat = edge_attr.reshape(E * 4)
    w1cb = jnp.concatenate([W1[2 * D:], b1.reshape(1, D)], axis=0)

    nblk = 1000
    xa, xb = pl.pallas_call(
        _proj_body,
        grid=(N // nblk,),
        in_specs=[
            pl.BlockSpec((nblk, D), lambda i: (i, 0)),
            pl.BlockSpec((2 * D + 4, D), lambda i: (0, 0)),
        ],
        out_specs=[
            pl.BlockSpec((nblk, D), lambda i: (i, 0)),
            pl.BlockSpec((nblk, D), lambda i: (i, 0)),
        ],
        out_shape=[
            jax.ShapeDtypeStruct((N, D), jnp.float32),
            jax.ShapeDtypeStruct((N, D), jnp.float32),
        ],
    )(x, W1)

    mesh = plsc.VectorSubcoreMesh(core_axis_name="c", subcore_axis_name="s")
    edge_loop = functools.partial(
        pl.kernel,
        mesh=mesh,
        out_type=[
            jax.ShapeDtypeStruct((NC, NR, D), jnp.float32),
            jax.ShapeDtypeStruct((NC, NR, CW), jnp.float32),
        ],
        scratch_types=[
            pltpu.VMEM((BB,), jnp.int32),
            pltpu.VMEM((BB,), jnp.int32),
            pltpu.VMEM((1, BB), jnp.int32),
            pltpu.VMEM((BB, D), jnp.float32),
            pltpu.VMEM((BB, D), jnp.float32),
            pltpu.VMEM((BB * 4,), jnp.float32),
            pltpu.VMEM((5, D), jnp.float32),
            pltpu.VMEM((BB, CW), jnp.float32),
            pltpu.VMEM((OC, CW), jnp.float32),
            pltpu.VMEM_SHARED((SR, D), jnp.float32),
            pltpu.VMEM_SHARED((SR, CW), jnp.float32),
            pltpu.SemaphoreType.DMA,
            pltpu.SemaphoreType.DMA,
        ],
    )(_edge_loop_body)
    s2, c2 = edge_loop(xa, xb, attr_flat, w1cb, src, dst)
    s_flat = s2.reshape(NC * NR, D)
    c_flat = c2.reshape(NC * NR, CW)

    out = pl.pallas_call(
        _node_body,
        grid=(N // nblk,),
        in_specs=[
            pl.BlockSpec((nblk, D), lambda i: (i, 0)),
            pl.BlockSpec((nblk, D), lambda i: (i, 0)),
            pl.BlockSpec((nblk, CW), lambda i: (i, 0)),
            pl.BlockSpec((D, D), lambda i: (0, 0)),
            pl.BlockSpec((1, D), lambda i: (0, 0)),
            pl.BlockSpec((2 * D, D), lambda i: (0, 0)),
            pl.BlockSpec((1, D), lambda i: (0, 0)),
            pl.BlockSpec((D, D), lambda i: (0, 0)),
            pl.BlockSpec((1, D), lambda i: (0, 0)),
        ],
        out_specs=pl.BlockSpec((nblk, D), lambda i: (i, 0)),
        out_shape=jax.ShapeDtypeStruct((N, D), jnp.float32),
    )(x, s_flat, c_flat, W2, b2.reshape(1, D), W3, b3.reshape(1, D),
      W4, b4.reshape(1, D))
    return out


# speedup vs baseline: 1.9109x; 1.1429x over previous
import functools

import jax
import jax.numpy as jnp
from jax import lax
from jax.experimental import pallas as pl
from jax.experimental.pallas import tpu as pltpu
from jax.experimental.pallas import tpu_sc as plsc

D = 128
NV = D // 16
N = 10000
E = 320000
NC = 2
NS = 16
EPT = E // NS
BB = 80
NB = EPT // BB
NR = 5120
TRASH = NR
SR = NR + 8
RPT = NR // NS
OC = 16
CW = 16


def _proj_body(x_ref, w1_ref, xa_ref, xb_ref):
    xv = x_ref[...]
    xa_ref[...] = jnp.dot(xv, w1_ref[0:D, :], preferred_element_type=jnp.float32)
    xb_ref[...] = jnp.dot(xv, w1_ref[D:2 * D, :], preferred_element_type=jnp.float32)


def _edge_loop_body(xa, xb, attr, w1cb, src_i, dst_i,
                    s_out, c_out,
                    idx_s, idx_d, idx_dl, buf_a, buf_b, abuf, wbuf,
                    ones_v, zbuf_c, s_sh, c_sh,
                    sem_a, sem_b, sem_e, sem_i, sem_j):
    cid = lax.axis_index("c")
    sid = lax.axis_index("s")

    zero16 = jnp.zeros((16,), jnp.float32)
    onerow = jnp.where(lax.iota(jnp.int32, 16) == 0,
                       jnp.float32(1.0), jnp.float32(0.0))

    def _zrow(i, _):
        for v in range(NV):
            buf_a[i, pl.ds(v * 16, 16)] = zero16
        zbuf_c[i, :] = zero16
        return 0
    lax.fori_loop(0, OC, _zrow, 0)

    def _orow(i, _):
        ones_v[i, :] = onerow
        return 0
    lax.fori_loop(0, BB, _orow, 0)

    def _zcp(i, _):
        row0 = sid * RPT + i * OC
        pltpu.sync_copy(buf_a.at[pl.ds(0, OC)], s_sh.at[pl.ds(row0, OC)])
        pltpu.sync_copy(zbuf_c.at[pl.ds(0, OC)], c_sh.at[pl.ds(row0, OC)])
        return 0
    lax.fori_loop(0, RPT // OC, _zcp, 0)
    plsc.subcore_barrier()

    roff = cid * NR
    trash16 = jnp.full((16,), TRASH, jnp.int32)

    pltpu.sync_copy(w1cb, wbuf)
    wv = [[wbuf[k, pl.ds(v * 16, 16)] for k in range(5)] for v in range(NV)]

    base = sid * EPT

    def _batch(j, _):
        eoff = base + j * BB
        cp_e = pltpu.async_copy(attr.at[pl.ds(eoff * 4, BB * 4)], abuf, sem_e)
        cp_i = pltpu.async_copy(src_i.at[pl.ds(eoff, BB)], idx_s, sem_i)
        cp_j = pltpu.async_copy(dst_i.at[pl.ds(eoff, BB)], idx_d, sem_j)
        cp_i.wait()
        cp_j.wait()
        cp_a = pltpu.async_copy(xa.at[idx_s], buf_a, sem_a)
        cp_b = pltpu.async_copy(xb.at[idx_d], buf_b, sem_b)
        cp_e.wait()
        for v in range(BB // 16):
            sl = pl.ds(v * 16, 16)
            lidx = idx_d[sl] - roff
            ok = (lidx >= 0) & (lidx < NR)
            idx_dl[0, sl] = jnp.where(ok, lidx, trash16)
        cp_a.wait()
        cp_b.wait()

        def _rowq(q, _):
            av = abuf[pl.ds(16 * q, 16)]
            for p in range(4):
                r = 4 * q + p
                s0, s1, s2, s3 = (av[4 * p], av[4 * p + 1],
                                  av[4 * p + 2], av[4 * p + 3])
                for v in range(NV):
                    sl = pl.ds(v * 16, 16)
                    t = (wv[v][4] + s0 * wv[v][0] + s1 * wv[v][1]
                         + s2 * wv[v][2] + s3 * wv[v][3])
                    buf_a[r, sl] = jnp.maximum(
                        buf_a[r, sl] + buf_b[r, sl] + t, jnp.float32(0.0))
            return 0
        lax.fori_loop(0, BB // 4, _rowq, 0)

        pltpu.sync_copy(buf_a, s_sh.at[idx_dl.at[0]], add=True)
        pltpu.sync_copy(ones_v, c_sh.at[idx_dl.at[0]], add=True)
        return 0
    lax.fori_loop(0, NB, _batch, 0)
    plsc.subcore_barrier()

    def _ocp(i, _):
        row0 = sid * RPT + i * OC
        pltpu.sync_copy(s_sh.at[pl.ds(row0, OC)], s_out.at[cid, pl.ds(row0, OC)])
        pltpu.sync_copy(c_sh.at[pl.ds(row0, OC)], c_out.at[cid, pl.ds(row0, OC)])
        return 0
    lax.fori_loop(0, RPT // OC, _ocp, 0)


def _node_body(x_ref, s_ref, c_ref, w2_ref, b2_ref, w3_ref, b3_ref,
               w4_ref, b4_ref, out_ref):
    xv = x_ref[...]
    cnt = c_ref[:, 0:1]
    agg = (jnp.dot(s_ref[...], w2_ref[...], preferred_element_type=jnp.float32)
           + cnt * b2_ref[...])
    hn = jnp.maximum(
        jnp.dot(xv, w3_ref[0:D, :], preferred_element_type=jnp.float32)
        + jnp.dot(agg, w3_ref[D:2 * D, :], preferred_element_type=jnp.float32)
        + b3_ref[...],
        jnp.float32(0.0))
    out_ref[...] = (jnp.dot(hn, w4_ref[...], preferred_element_type=jnp.float32)
                    + b4_ref[...] + xv)


def kernel(x, edge_index, edge_attr, W1, b1, W2, b2, W3, b3, W4, b4):
    src = edge_index[0].astype(jnp.int32)
    dst = edge_index[1].astype(jnp.int32)
    attr_flat = edge_attr.reshape(E * 4)
    w1cb = jnp.concatenate([W1[2 * D:], b1.reshape(1, D)], axis=0)

    nblk = 1000
    xa, xb = pl.pallas_call(
        _proj_body,
        grid=(N // nblk,),
        in_specs=[
            pl.BlockSpec((nblk, D), lambda i: (i, 0)),
            pl.BlockSpec((2 * D + 4, D), lambda i: (0, 0)),
        ],
        out_specs=[
            pl.BlockSpec((nblk, D), lambda i: (i, 0)),
            pl.BlockSpec((nblk, D), lambda i: (i, 0)),
        ],
        out_shape=[
            jax.ShapeDtypeStruct((N, D), jnp.float32),
            jax.ShapeDtypeStruct((N, D), jnp.float32),
        ],
    )(x, W1)

    mesh = plsc.VectorSubcoreMesh(core_axis_name="c", subcore_axis_name="s")
    edge_loop = functools.partial(
        pl.kernel,
        mesh=mesh,
        out_type=[
            jax.ShapeDtypeStruct((NC, NR, D), jnp.float32),
            jax.ShapeDtypeStruct((NC, NR, CW), jnp.float32),
        ],
        scratch_types=[
            pltpu.VMEM((BB,), jnp.int32),
            pltpu.VMEM((BB,), jnp.int32),
            pltpu.VMEM((1, BB), jnp.int32),
            pltpu.VMEM((BB, D), jnp.float32),
            pltpu.VMEM((BB, D), jnp.float32),
            pltpu.VMEM((BB * 4,), jnp.float32),
            pltpu.VMEM((5, D), jnp.float32),
            pltpu.VMEM((BB, CW), jnp.float32),
            pltpu.VMEM((OC, CW), jnp.float32),
            pltpu.VMEM_SHARED((SR, D), jnp.float32),
            pltpu.VMEM_SHARED((SR, CW), jnp.float32),
        ] + [pltpu.SemaphoreType.DMA] * 5,
    )(_edge_loop_body)
    s2, c2 = edge_loop(xa, xb, attr_flat, w1cb, src, dst)
    s_flat = s2.reshape(NC * NR, D)
    c_flat = c2.reshape(NC * NR, CW)

    out = pl.pallas_call(
        _node_body,
        grid=(N // nblk,),
        in_specs=[
            pl.BlockSpec((nblk, D), lambda i: (i, 0)),
            pl.BlockSpec((nblk, D), lambda i: (i, 0)),
            pl.BlockSpec((nblk, CW), lambda i: (i, 0)),
            pl.BlockSpec((D, D), lambda i: (0, 0)),
            pl.BlockSpec((1, D), lambda i: (0, 0)),
            pl.BlockSpec((2 * D, D), lambda i: (0, 0)),
            pl.BlockSpec((1, D), lambda i: (0, 0)),
            pl.BlockSpec((D, D), lambda i: (0, 0)),
            pl.BlockSpec((1, D), lambda i: (0, 0)),
        ],
        out_specs=pl.BlockSpec((nblk, D), lambda i: (i, 0)),
        out_shape=jax.ShapeDtypeStruct((N, D), jnp.float32),
    )(x, s_flat, c_flat, W2, b2.reshape(1, D), W3, b3.reshape(1, D),
      W4, b4.reshape(1, D))
    return out
